# baseline (speedup 1.0000x reference)
import functools

import jax
import jax.numpy as jnp
from jax import lax
from jax.experimental import pallas as pl
from jax.experimental.pallas import tpu as pltpu
from jax.experimental.pallas import tpu_sc as plsc

H = 128
SB = 40
NTILE = 16
NSC = 2
EBATCH = 128
ZROWS = 816


def _ceil_to(x, m):
    return -(-x // m) * m


def _sc_segmean(table, sidx, gidx, zeros_big, ones_big, resid, e_real,
                n_seg, n_chunks, n_resid):
    e2 = sidx.shape[0]
    ept = e2 // NTILE
    nbatch = ept // EBATCH
    n2 = n_seg // NSC
    ch = -(-n2 // n_chunks)
    cht = _ceil_to(-(-ch // NTILE), SB)
    chpad = NTILE * cht
    n2pad = n_chunks * chpad
    npad = NSC * n2pad
    gather = gidx is not None
    has_resid = resid is not None

    def body(*refs):
        it = iter(refs)
        table_r = next(it)
        sidx_r = next(it)
        gidx_r = next(it) if gather else None
        resid_r = next(it) if has_resid else None
        zeros_r = next(it)
        ones_r = next(it)
        out_r = next(it)
        acc = next(it)
        cnt = next(it)
        dbuf = next(it)
        gbuf = next(it) if gather else None
        sidxv = next(it)
        rows = next(it)
        ones = next(it)
        sbuf = next(it)
        cbuf = next(it)
        rbuf = next(it) if has_resid else None
        gsem = next(it)

        c = lax.axis_index("c")
        t = lax.axis_index("s")

        pltpu.sync_copy(ones_r.at[pl.ds(0, EBATCH)], ones)

        for ci in range(n_chunks):
            lo = c * n2pad + ci * chpad
            pltpu.sync_copy(zeros_r.at[pl.ds(0, cht)], acc.at[pl.ds(t * cht, cht)])
            pltpu.sync_copy(zeros_r.at[pl.ds(0, cht)], cnt.at[pl.ds(t * cht, cht)])

            @pl.when(t == 0)
            def _zero_dump():
                pltpu.sync_copy(zeros_r.at[pl.ds(0, 16)], acc.at[pl.ds(chpad, 16)])
                pltpu.sync_copy(zeros_r.at[pl.ds(0, 16)], cnt.at[pl.ds(chpad, 16)])

            plsc.subcore_barrier()

            def _edge(b, carry):
                base = t * ept + b * EBATCH
                pltpu.sync_copy(sidx_r.at[pl.ds(base, EBATCH)], dbuf)
                if gather:
                    pltpu.sync_copy(gidx_r.at[pl.ds(base, EBATCH)], gbuf)
                    cp = pltpu.async_copy(table_r.at[gbuf], rows, gsem)
                else:
                    cp = pltpu.async_copy(table_r.at[pl.ds(base, EBATCH)], rows, gsem)
                for k in range(EBATCH // 16):
                    d = dbuf[pl.ds(k * 16, 16)]
                    eid = base + k * 16 + lax.iota(jnp.int32, 16)
                    m = (eid < e_real) & (d >= lo) & (d < lo + chpad)
                    sidxv[pl.ds(k * 16, 16)] = jnp.where(m, d - lo, chpad)
                cp.wait()
                pltpu.sync_copy(rows, acc.at[sidxv], add=True)
                pltpu.sync_copy(ones, cnt.at[sidxv], add=True)
                return carry
            lax.fori_loop(0, nbatch, _edge, 0)

            plsc.subcore_barrier()

            def _out(sbi, carry):
                r0 = t * cht + sbi * SB
                pltpu.sync_copy(acc.at[pl.ds(r0, SB)], sbuf)
                pltpu.sync_copy(cnt.at[pl.ds(r0, SB)], cbuf)
                if has_resid:
                    roff = lo + r0
                    if npad > n_resid:
                        roff = jnp.minimum(roff, n_resid - SB)
                    pltpu.sync_copy(resid_r.at[pl.ds(roff, SB)], rbuf)

                def _row(r, carry2):
                    for k in range(H // 16):
                        cv = jnp.maximum(cbuf[r, pl.ds(k * 16, 16)], 1.0)
                        v = sbuf[r, pl.ds(k * 16, 16)] / cv
                        if has_resid:
                            v = v + rbuf[r, pl.ds(k * 16, 16)]
                        sbuf[r, pl.ds(k * 16, 16)] = v
                    return carry2
                lax.fori_loop(0, SB, _row, 0)
                pltpu.sync_copy(sbuf, out_r.at[pl.ds(lo + r0, SB)])
                return carry
            lax.fori_loop(0, cht // SB, _out, 0)

    scratch = [
        pltpu.VMEM_SHARED((chpad + 16, H), jnp.float32),
        pltpu.VMEM_SHARED((chpad + 16, H), jnp.float32),
        pltpu.VMEM((EBATCH,), jnp.int32),
    ]
    if gather:
        scratch.append(pltpu.VMEM((EBATCH,), jnp.int32))
    scratch += [
        pltpu.VMEM((EBATCH,), jnp.int32),
        pltpu.VMEM((EBATCH, H), jnp.float32),
        pltpu.VMEM((EBATCH, H), jnp.float32),
        pltpu.VMEM((SB, H), jnp.float32),
        pltpu.VMEM((SB, H), jnp.float32),
    ]
    if has_resid:
        scratch.append(pltpu.VMEM((SB, H), jnp.float32))
    scratch.append(pltpu.SemaphoreType.DMA)

    mesh = plsc.VectorSubcoreMesh(core_axis_name="c", subcore_axis_name="s",
                                  num_cores=NSC, num_subcores=NTILE)
    fn = pl.kernel(body,
                   out_type=jax.ShapeDtypeStruct((npad, H), jnp.float32),
                   mesh=mesh, scratch_types=scratch)
    args = [table, sidx]
    if gather:
        args.append(gidx)
    if has_resid:
        args.append(resid)
    args += [zeros_big, ones_big]
    return fn(*args)


def _temporal_body(ts_ref, w1_ref, b1_ref, w2_ref, b2_ref, o_ref):
    t = ts_ref[...]
    h = jnp.maximum(t * w1_ref[...] + b1_ref[...], 0.0)
    et = lax.dot_general(h, w2_ref[...], (((1,), (1,)), ((), ())),
                         preferred_element_type=jnp.float32)
    o_ref[...] = et + b2_ref[...]


def _tc_temporal(ts, w1, b1, w2, b2, e2):
    e = ts.shape[0]
    bb = 1000
    grid = e // bb
    return pl.pallas_call(
        _temporal_body,
        grid=(grid,),
        in_specs=[
            pl.BlockSpec((bb, 1), lambda i: (i, 0)),
            pl.BlockSpec((1, 32), lambda i: (0, 0)),
            pl.BlockSpec((1, 32), lambda i: (0, 0)),
            pl.BlockSpec((H, 32), lambda i: (0, 0)),
            pl.BlockSpec((1, H), lambda i: (0, 0)),
        ],
        out_specs=pl.BlockSpec((bb, H), lambda i: (i, 0)),
        out_shape=jax.ShapeDtypeStruct((e2, H), jnp.float32),
    )(ts.reshape(e, 1), w1.reshape(1, 32), b1.reshape(1, 32), w2,
      b2.reshape(1, H))


def _vote_body(a_ref, w1_ref, b1_ref, w2_ref, b2_ref, o_ref):
    a = a_ref[...]
    pol = jnp.clip(a[:, 0:1], 0.0, 1.0)
    h = lax.dot_general(a, w1_ref[...], (((1,), (1,)), ((), ())),
                        preferred_element_type=jnp.float32)
    h = jnp.maximum(h + b1_ref[...], 0.0)
    e = lax.dot_general(h, w2_ref[...], (((1,), (1,)), ((), ())),
                        preferred_element_type=jnp.float32)
    o_ref[...] = (e + b2_ref[...]) * (pol + 0.01)


def _tc_vote(attr, w1, b1, w2, b2, e2):
    ev, aw = attr.shape
    bb = 400
    grid = ev // bb
    w1e = jnp.concatenate([jnp.zeros((H, 1), jnp.float32), w1], axis=1)
    return pl.pallas_call(
        _vote_body,
        grid=(grid,),
        in_specs=[
            pl.BlockSpec((bb, aw), lambda i: (i, 0)),
            pl.BlockSpec((H, aw), lambda i: (0, 0)),
            pl.BlockSpec((1, H), lambda i: (0, 0)),
            pl.BlockSpec((H, H), lambda i: (0, 0)),
            pl.BlockSpec((1, H), lambda i: (0, 0)),
        ],
        out_specs=pl.BlockSpec((bb, H), lambda i: (i, 0)),
        out_shape=jax.ShapeDtypeStruct((e2, H), jnp.float32),
    )(attr, w1e, b1.reshape(1, H), w2, b2.reshape(1, H))


def _l2norm(v):
    n = jnp.sqrt(jnp.sum(v * v, axis=1, keepdims=True))
    return v / jnp.maximum(n, 1e-12)


def _leg_body(ad_ref, al_ref, xl_ref, dwl_ref, dbl_ref, dwr_ref,
              lwl_ref, lbl_ref, lwr_ref, o_ref):
    xl = xl_ref[...]
    cd = lax.dot_general(ad_ref[...], dwl_ref[...], (((1,), (1,)), ((), ())),
                         preferred_element_type=jnp.float32) + dbl_ref[...]
    cd = cd + lax.dot_general(xl, dwr_ref[...], (((1,), (1,)), ((), ())),
                              preferred_element_type=jnp.float32)
    cl = lax.dot_general(al_ref[...], lwl_ref[...], (((1,), (1,)), ((), ())),
                         preferred_element_type=jnp.float32) + lbl_ref[...]
    cl = cl + lax.dot_general(xl, lwr_ref[...], (((1,), (1,)), ((), ())),
                              preferred_element_type=jnp.float32)
    o_ref[...] = _l2norm(cd) + _l2norm(cl) + xl


def _tc_leg(aggr_don, aggr_lob, x_leg, dwl, dbl, dwr, lwl, lbl, lwr):
    nl = x_leg.shape[0]
    bb = 200
    grid = nl // bb
    full = lambda i: (0, 0)
    row = lambda i: (i, 0)
    return pl.pallas_call(
        _leg_body,
        grid=(grid,),
        in_specs=[
            pl.BlockSpec((bb, H), row),
            pl.BlockSpec((bb, H), row),
            pl.BlockSpec((bb, H), row),
            pl.BlockSpec((H, H), full), pl.BlockSpec((1, H), full),
            pl.BlockSpec((H, H), full),
            pl.BlockSpec((H, H), full), pl.BlockSpec((1, H), full),
            pl.BlockSpec((H, H), full),
        ],
        out_specs=pl.BlockSpec((bb, H), row),
        out_shape=jax.ShapeDtypeStruct((nl, H), jnp.float32),
    )(aggr_don, aggr_lob, x_leg, dwl, dbl.reshape(1, H), dwr,
      lwl, lbl.reshape(1, H), lwr)


def _bill_body(av_ref, xb_ref, wl_ref, bl_ref, wr_ref, o_ref):
    xb = xb_ref[...]
    cv = lax.dot_general(av_ref[...], wl_ref[...], (((1,), (1,)), ((), ())),
                         preferred_element_type=jnp.float32) + bl_ref[...]
    cv = cv + lax.dot_general(xb, wr_ref[...], (((1,), (1,)), ((), ())),
                              preferred_element_type=jnp.float32)
    o_ref[...] = _l2norm(cv) + xb


def _tc_bill(aggr_ver, x_bill, wl, bl, wr):
    nb = x_bill.shape[0]
    bb = 400
    grid = nb // bb
    full = lambda i: (0, 0)
    row = lambda i: (i, 0)
    return pl.pallas_call(
        _bill_body,
        grid=(grid,),
        in_specs=[
            pl.BlockSpec((bb, H), row),
            pl.BlockSpec((bb, H), row),
            pl.BlockSpec((H, H), full), pl.BlockSpec((1, H), full),
            pl.BlockSpec((H, H), full),
        ],
        out_specs=pl.BlockSpec((bb, H), row),
        out_shape=jax.ShapeDtypeStruct((nb, H), jnp.float32),
    )(aggr_ver, x_bill, wl, bl.reshape(1, H), wr)


def _pad_idx(a, e2):
    return jnp.pad(a, (0, e2 - a.shape[0]))


def kernel(x_donor, x_leg, x_lobby, x_bv, x_bill, vote_attr, ts_donated,
           ts_lobbied, dW1, db1, dW2, db2, lW1, lb1, lW2, lb2,
           don_Wl, don_bl, don_Wr, lob_Wl, lob_bl, lob_Wr,
           ver_Wl, ver_bl, ver_Wr, vote_W1, vote_b1, vote_W2, vote_b2,
           donated_src, donated_dst, lobbied_src, lobbied_dst,
           isver_src, isver_dst, voted_src, voted_dst):
    nd = x_donor.shape[0]
    nl = x_leg.shape[0]
    nf = x_lobby.shape[0]
    nv = x_bv.shape[0]
    nb = x_bill.shape[0]
    ed = donated_src.shape[0]
    el = lobbied_src.shape[0]
    ei = isver_src.shape[0]
    ev = voted_src.shape[0]

    e2d = _ceil_to(ed, NTILE * EBATCH)
    e2l = _ceil_to(el, NTILE * EBATCH)
    e2i = _ceil_to(ei, NTILE * EBATCH)
    e2v = _ceil_to(ev, NTILE * EBATCH)

    zeros_big = jnp.zeros((ZROWS, H), jnp.float32)
    ones_big = jnp.ones((ZROWS, H), jnp.float32)

    et_don = _tc_temporal(ts_donated, dW1, db1, dW2, db2, e2d)
    et_lob = _tc_temporal(ts_lobbied, lW1, lb1, lW2, lb2, e2l)
    e_vote = _tc_vote(vote_attr, vote_W1, vote_b1, vote_W2, vote_b2, e2v)

    xs_don = _sc_segmean(et_don, _pad_idx(donated_src, e2d), None,
                         zeros_big, ones_big, x_donor, ed, nd, 6, nd)
    xs_lob = _sc_segmean(et_lob, _pad_idx(lobbied_src, e2l), None,
                         zeros_big, ones_big, x_lobby, el, nf, 3, nf)
    out3 = _sc_segmean(e_vote, _pad_idx(voted_dst, e2v), None,
                       zeros_big, ones_big, x_bv, ev, nv, 12, nv)[:nv]
    aggr_ver = _sc_segmean(x_bv, _pad_idx(isver_dst, e2i),
                           _pad_idx(isver_src, e2i),
                           zeros_big, ones_big, None, ei, nb, 3, 0)
    aggr_don = _sc_segmean(xs_don, _pad_idx(donated_dst, e2d),
                           _pad_idx(donated_src, e2d),
                           zeros_big, ones_big, None, ed, nl, 1, 0)
    aggr_lob = _sc_segmean(xs_lob, _pad_idx(lobbied_dst, e2l),
                           _pad_idx(lobbied_src, e2l),
                           zeros_big, ones_big, None, el, nl, 1, 0)

    out1 = _tc_leg(aggr_don[:nl] if aggr_don.shape[0] != nl else aggr_don,
                   aggr_lob[:nl] if aggr_lob.shape[0] != nl else aggr_lob,
                   x_leg, don_Wl, don_bl, don_Wr, lob_Wl, lob_bl, lob_Wr)
    out4 = _tc_bill(aggr_ver[:nb] if aggr_ver.shape[0] != nb else aggr_ver,
                    x_bill, ver_Wl, ver_bl, ver_Wr)

    return x_donor, out1, x_lobby, out3, out4
